# baseline (speedup 1.0000x reference)
import functools

import jax
import jax.numpy as jnp
from jax import lax
from jax.experimental import pallas as pl
from jax.experimental.pallas import tpu as pltpu
from jax.experimental.pallas import tpu_sc as plsc

NC = 2
NS = 16
KD = 80
BN = 1000
V_TH = 1.0
T = 4


def _deg_body(dst_hbm, zeros_hbm, degp_hbm, idx_v, ones_v, deg_sh, sem):
    nchunk = dst_hbm.shape[2]
    c = lax.axis_index("c")
    s = lax.axis_index("s")

    @pl.when(s == 0)
    def _():
        pltpu.sync_copy(zeros_hbm, deg_sh)

    for i in range(KD // 16):
        ones_v[pl.ds(i * 16, 16)] = jnp.ones((16,), jnp.float32)
    pltpu.sync_copy(dst_hbm.at[c, s], idx_v)
    plsc.subcore_barrier()

    def body(j, carry):
        pltpu.sync_copy(ones_v, deg_sh.at[idx_v.at[j]], add=True)
        return carry

    lax.fori_loop(0, nchunk, body, 0)
    plsc.subcore_barrier()

    @pl.when(s == 0)
    def _():
        pltpu.sync_copy(deg_sh, degp_hbm.at[c])


def _deg_partials(dst_r, zeros_n, n):
    nchunk = dst_r.shape[2]
    kern = pl.kernel(
        _deg_body,
        out_type=jax.ShapeDtypeStruct((NC, n), jnp.float32),
        mesh=plsc.VectorSubcoreMesh(core_axis_name="c", subcore_axis_name="s"),
        scratch_types=[
            pltpu.VMEM((nchunk, KD), jnp.int32),
            pltpu.VMEM((KD,), jnp.float32),
            pltpu.MemorySpace.VMEM_SHARED((n,), jnp.float32),
            pltpu.SemaphoreType.DMA,
        ],
    )
    return kern(dst_r, zeros_n)


def _spmem_slices(s, n, src_at, dst_at):
    rpt8 = (n // NS // 8) * 8
    tail = n - NS * rpt8
    pltpu.sync_copy(src_at(s * rpt8, rpt8), dst_at(s * rpt8, rpt8))
    if tail:
        @pl.when(s == 0)
        def _():
            pltpu.sync_copy(src_at(NS * rpt8, tail), dst_at(NS * rpt8, tail))


def _scatter_body(g_hbm, eidx_hbm, zeros_hbm, accp_hbm,
                  idx_v, rows_v, acc_sh, sems_i, sems_r):
    nch = eidx_hbm.shape[2] - 4
    np_ = zeros_hbm.shape[0]
    n = accp_hbm.shape[1]
    c = lax.axis_index("c")
    s = lax.axis_index("s")

    _spmem_slices(s, np_, lambda o, l: zeros_hbm.at[pl.ds(o, l)],
                  lambda o, l: acc_sh.at[pl.ds(o, l)])
    plsc.subcore_barrier()

    for q in range(4):
        pltpu.async_copy(eidx_hbm.at[c, s, q], idx_v[q], sems_i[q])
    pltpu.make_async_copy(eidx_hbm.at[c, s, 0], idx_v[0], sems_i[0]).wait()
    pltpu.async_copy(g_hbm.at[idx_v[0].at[0]], rows_v[0], sems_r[0])

    def _step(a, qc, qn):
        pltpu.make_async_copy(eidx_hbm.at[c, s, a + 1],
                              idx_v[qn], sems_i[qn]).wait()
        pltpu.async_copy(g_hbm.at[idx_v[qn].at[0]],
                         rows_v[qn % 2], sems_r[qn % 2])
        pltpu.make_async_copy(g_hbm.at[idx_v[qc].at[0]],
                              rows_v[qc % 2], sems_r[qc % 2]).wait()
        pltpu.sync_copy(rows_v[qc % 2], acc_sh.at[idx_v[qc].at[1]], add=True)
        pltpu.async_copy(eidx_hbm.at[c, s, a + 4], idx_v[qc], sems_i[qc])

    def body(i, carry):
        for q in range(4):
            _step(4 * i + q, q, (q + 1) % 4)
        return carry

    lax.fori_loop(0, nch // 4, body, 0)
    pltpu.make_async_copy(g_hbm.at[idx_v[0].at[0]], rows_v[0],
                          sems_r[0]).wait()
    for q in range(1, 4):
        pltpu.make_async_copy(eidx_hbm.at[c, s, q], idx_v[q],
                              sems_i[q]).wait()
    plsc.subcore_barrier()
    _spmem_slices(s, n, lambda o, l: acc_sh.at[pl.ds(o, l)],
                  lambda o, l: accp_hbm.at[c, pl.ds(o, l)])


def _scatter_partials(g, eidx, zeros_np, n, d):
    np_ = zeros_np.shape[0]
    k2 = eidx.shape[4]
    kern = pl.kernel(
        _scatter_body,
        out_type=jax.ShapeDtypeStruct((NC, n, d), jnp.float32),
        mesh=plsc.VectorSubcoreMesh(core_axis_name="c", subcore_axis_name="s"),
        scratch_types=[
            [pltpu.VMEM((2, k2), jnp.int32) for _ in range(4)],
            [pltpu.VMEM((k2, d), jnp.float32) for _ in range(2)],
            pltpu.MemorySpace.VMEM_SHARED((np_, d), jnp.float32),
            [pltpu.SemaphoreType.DMA for _ in range(4)],
            [pltpu.SemaphoreType.DMA for _ in range(2)],
        ],
    )
    return kern(g, eidx, zeros_np)


def _dinv_of(degp_blk):
    deg = degp_blk[:, 0:1] + degp_blk[:, 1:2] + 1.0
    return lax.rsqrt(jnp.maximum(deg, 1e-12))


def _mm_scale_body(x_ref, w_ref, degp_ref, g_ref):
    h = jnp.dot(x_ref[...], w_ref[...], preferred_element_type=jnp.float32)
    g_ref[...] = _dinv_of(degp_ref[...]) * h


def _mm_scale(x, w, degp_t):
    n, din = x.shape
    dout = w.shape[1]
    return pl.pallas_call(
        _mm_scale_body,
        grid=(n // BN,),
        in_specs=[
            pl.BlockSpec((BN, din), lambda i: (i, 0)),
            pl.BlockSpec((din, dout), lambda i: (0, 0)),
            pl.BlockSpec((BN, NC), lambda i: (i, 0)),
        ],
        out_specs=pl.BlockSpec((BN, dout), lambda i: (i, 0)),
        out_shape=jax.ShapeDtypeStruct((n, dout), jnp.float32),
    )(x, w, degp_t)


def _if_body(accp_ref, g_ref, degp_ref, b_ref, o_ref, z_ref):
    dinv = _dinv_of(degp_ref[...])
    g = g_ref[...]
    y = dinv * (accp_ref[0] + accp_ref[1] + g) + b_ref[...]
    z = jnp.zeros_like(y)
    for t in range(T):
        z = z + y
        o = (z >= V_TH).astype(jnp.float32)
        z = z * (1.0 - o)
        o_ref[t] = o
        z_ref[t] = z


def _if_dynamics(accp, g, degp_t, b2d):
    n, d = g.shape
    out_sds = jax.ShapeDtypeStruct((T, n, d), jnp.float32)
    return pl.pallas_call(
        _if_body,
        grid=(n // BN,),
        in_specs=[
            pl.BlockSpec((NC, BN, d), lambda i: (0, i, 0)),
            pl.BlockSpec((BN, d), lambda i: (i, 0)),
            pl.BlockSpec((BN, NC), lambda i: (i, 0)),
            pl.BlockSpec((1, d), lambda i: (0, 0)),
        ],
        out_specs=[
            pl.BlockSpec((T, BN, d), lambda i: (0, i, 0)),
            pl.BlockSpec((T, BN, d), lambda i: (0, i, 0)),
        ],
        out_shape=[out_sds, out_sds],
    )(accp, g, degp_t, b2d)


def kernel(x, edge_index, W, b):
    n, din = x.shape
    dout = W.shape[1]
    e = edge_index.shape[1]
    ept = e // (NC * NS)
    nchunk_d = ept // KD
    k2 = 128
    nch = ((-(-ept // k2) + 3) // 4) * 4
    npad = nch * k2 - ept
    np_ = n + 8

    src_t = edge_index[0].reshape(NC, NS, ept)
    dst_t = edge_index[1].reshape(NC, NS, ept)
    srcp = jnp.concatenate(
        [src_t, jnp.zeros((NC, NS, npad), jnp.int32)], axis=2)
    dstp = jnp.concatenate(
        [dst_t, jnp.full((NC, NS, npad), n, jnp.int32)], axis=2)
    eidx = jnp.stack([srcp.reshape(NC, NS, nch, k2),
                      dstp.reshape(NC, NS, nch, k2)], axis=3)
    chunk_pad = jnp.concatenate(
        [jnp.zeros((NC, NS, 4, 1, k2), jnp.int32),
         jnp.full((NC, NS, 4, 1, k2), n, jnp.int32)], axis=3)
    eidx = jnp.concatenate([eidx, chunk_pad], axis=2)

    dst_rd = edge_index[1].reshape(NC, NS, nchunk_d, KD)
    zeros_n = jnp.zeros((n,), jnp.float32)
    zeros_np = jnp.zeros((np_, dout), jnp.float32)

    degp = _deg_partials(dst_rd, zeros_n, n)
    degp_t = degp.T
    g = _mm_scale(x, W, degp_t)
    accp = _scatter_partials(g, eidx, zeros_np, n, dout)
    o_seq, z_seq = _if_dynamics(accp, g, degp_t, b.reshape(1, dout))
    return (o_seq, z_seq)


# speedup vs baseline: 1.0970x; 1.0970x over previous
import functools

import jax
import jax.numpy as jnp
from jax import lax
from jax.experimental import pallas as pl
from jax.experimental.pallas import tpu as pltpu
from jax.experimental.pallas import tpu_sc as plsc

NC = 2
NS = 16
KD = 80
BN = 1000
V_TH = 1.0
T = 4


def _deg_body(dst_hbm, zeros_hbm, degp_hbm, idx_v, ones_v, deg_sh, sem):
    nchunk = dst_hbm.shape[2]
    c = lax.axis_index("c")
    s = lax.axis_index("s")

    @pl.when(s == 0)
    def _():
        pltpu.sync_copy(zeros_hbm, deg_sh)

    for i in range(KD // 16):
        ones_v[pl.ds(i * 16, 16)] = jnp.ones((16,), jnp.float32)
    pltpu.sync_copy(dst_hbm.at[c, s], idx_v)
    plsc.subcore_barrier()

    def body(j, carry):
        pltpu.sync_copy(ones_v, deg_sh.at[idx_v.at[j]], add=True)
        return carry

    lax.fori_loop(0, nchunk, body, 0)
    plsc.subcore_barrier()

    @pl.when(s == 0)
    def _():
        pltpu.sync_copy(deg_sh, degp_hbm.at[c])


def _deg_partials(dst_r, zeros_n, n):
    nchunk = dst_r.shape[2]
    kern = pl.kernel(
        _deg_body,
        out_type=jax.ShapeDtypeStruct((NC, n), jnp.float32),
        mesh=plsc.VectorSubcoreMesh(core_axis_name="c", subcore_axis_name="s"),
        scratch_types=[
            pltpu.VMEM((nchunk, KD), jnp.int32),
            pltpu.VMEM((KD,), jnp.float32),
            pltpu.MemorySpace.VMEM_SHARED((n,), jnp.float32),
            pltpu.SemaphoreType.DMA,
        ],
    )
    return kern(dst_r, zeros_n)


def _spmem_slices(s, n, src_at, dst_at):
    rpt8 = (n // NS // 8) * 8
    tail = n - NS * rpt8
    pltpu.sync_copy(src_at(s * rpt8, rpt8), dst_at(s * rpt8, rpt8))
    if tail:
        @pl.when(s == 0)
        def _():
            pltpu.sync_copy(src_at(NS * rpt8, tail), dst_at(NS * rpt8, tail))


def _scatter_body(g_hbm, src_hbm, dst_hbm, zeros_hbm, accp_hbm,
                  sidx_v, didx_v, rows_a, rows_b, acc_sh, sem_a, sem_b):
    nch = dst_hbm.shape[2]
    ks = dst_hbm.shape[3]
    np_ = zeros_hbm.shape[0]
    n = accp_hbm.shape[1]
    c = lax.axis_index("c")
    s = lax.axis_index("s")

    _spmem_slices(s, np_, lambda o, l: zeros_hbm.at[pl.ds(o, l)],
                  lambda o, l: acc_sh.at[pl.ds(o, l)])
    pltpu.sync_copy(src_hbm.at[c, s], sidx_v)
    pltpu.sync_copy(dst_hbm.at[c, s], didx_v)
    plsc.subcore_barrier()

    def _gather(j, rows, sem):
        off = pl.multiple_of(j * ks, 8)
        return pltpu.async_copy(g_hbm.at[sidx_v.at[pl.ds(off, ks)]],
                                rows, sem)

    _gather(0, rows_b, sem_b).wait()

    def body(i, carry):
        a = 2 * i
        cpa = _gather(a + 1, rows_a, sem_a)
        pltpu.sync_copy(rows_b, acc_sh.at[didx_v.at[a]], add=True)
        cpa.wait()
        cpb = _gather(a + 2, rows_b, sem_b)
        pltpu.sync_copy(rows_a, acc_sh.at[didx_v.at[a + 1]], add=True)
        cpb.wait()
        return carry

    lax.fori_loop(0, nch // 2, body, 0)
    plsc.subcore_barrier()
    _spmem_slices(s, n, lambda o, l: acc_sh.at[pl.ds(o, l)],
                  lambda o, l: accp_hbm.at[c, pl.ds(o, l)])


def _scatter_partials(g, src_flat, dst_chunks, zeros_np, n, d):
    np_ = zeros_np.shape[0]
    nch = dst_chunks.shape[2]
    ks = dst_chunks.shape[3]
    kern = pl.kernel(
        _scatter_body,
        out_type=jax.ShapeDtypeStruct((NC, n, d), jnp.float32),
        mesh=plsc.VectorSubcoreMesh(core_axis_name="c", subcore_axis_name="s"),
        scratch_types=[
            pltpu.VMEM(((nch + 1) * ks,), jnp.int32),
            pltpu.VMEM((nch, ks), jnp.int32),
            pltpu.VMEM((ks, d), jnp.float32),
            pltpu.VMEM((ks, d), jnp.float32),
            pltpu.MemorySpace.VMEM_SHARED((np_, d), jnp.float32),
            pltpu.SemaphoreType.DMA,
            pltpu.SemaphoreType.DMA,
        ],
    )
    return kern(g, src_flat, dst_chunks, zeros_np)


def _dinv_of(degp_blk):
    deg = degp_blk[:, 0:1] + degp_blk[:, 1:2] + 1.0
    return lax.rsqrt(jnp.maximum(deg, 1e-12))


def _mm_scale_body(x_ref, w_ref, degp_ref, g_ref):
    h = jnp.dot(x_ref[...], w_ref[...], preferred_element_type=jnp.float32)
    g_ref[...] = _dinv_of(degp_ref[...]) * h


def _mm_scale(x, w, degp_t):
    n, din = x.shape
    dout = w.shape[1]
    return pl.pallas_call(
        _mm_scale_body,
        grid=(n // BN,),
        in_specs=[
            pl.BlockSpec((BN, din), lambda i: (i, 0)),
            pl.BlockSpec((din, dout), lambda i: (0, 0)),
            pl.BlockSpec((BN, NC), lambda i: (i, 0)),
        ],
        out_specs=pl.BlockSpec((BN, dout), lambda i: (i, 0)),
        out_shape=jax.ShapeDtypeStruct((n, dout), jnp.float32),
    )(x, w, degp_t)


def _if_body(accp_ref, g_ref, degp_ref, b_ref, o_ref, z_ref):
    dinv = _dinv_of(degp_ref[...])
    g = g_ref[...]
    y = dinv * (accp_ref[0] + accp_ref[1] + g) + b_ref[...]
    z = jnp.zeros_like(y)
    for t in range(T):
        z = z + y
        o = (z >= V_TH).astype(jnp.float32)
        z = z * (1.0 - o)
        o_ref[t] = o
        z_ref[t] = z


def _if_dynamics(accp, g, degp_t, b2d):
    n, d = g.shape
    out_sds = jax.ShapeDtypeStruct((T, n, d), jnp.float32)
    return pl.pallas_call(
        _if_body,
        grid=(n // BN,),
        in_specs=[
            pl.BlockSpec((NC, BN, d), lambda i: (0, i, 0)),
            pl.BlockSpec((BN, d), lambda i: (i, 0)),
            pl.BlockSpec((BN, NC), lambda i: (i, 0)),
            pl.BlockSpec((1, d), lambda i: (0, 0)),
        ],
        out_specs=[
            pl.BlockSpec((T, BN, d), lambda i: (0, i, 0)),
            pl.BlockSpec((T, BN, d), lambda i: (0, i, 0)),
        ],
        out_shape=[out_sds, out_sds],
    )(accp, g, degp_t, b2d)


def kernel(x, edge_index, W, b):
    n, din = x.shape
    dout = W.shape[1]
    e = edge_index.shape[1]
    ept = e // (NC * NS)
    nchunk_d = ept // KD
    ks = 104
    nch = -(-ept // ks)
    nch += nch % 2
    np_ = n + 8

    src_t = edge_index[0].reshape(NC, NS, ept)
    dst_t = edge_index[1].reshape(NC, NS, ept)
    src_flat = jnp.concatenate(
        [src_t, jnp.zeros((NC, NS, (nch + 1) * ks - ept), jnp.int32)],
        axis=2)
    dst_chunks = jnp.concatenate(
        [dst_t, jnp.full((NC, NS, nch * ks - ept), n, jnp.int32)],
        axis=2).reshape(NC, NS, nch, ks)

    dst_rd = edge_index[1].reshape(NC, NS, nchunk_d, KD)
    zeros_n = jnp.zeros((n,), jnp.float32)
    zeros_np = jnp.zeros((np_, dout), jnp.float32)

    degp = _deg_partials(dst_rd, zeros_n, n)
    degp_t = degp.T
    g = _mm_scale(x, W, degp_t)
    accp = _scatter_partials(g, src_flat, dst_chunks, zeros_np, n, dout)
    o_seq, z_seq = _if_dynamics(accp, g, degp_t, b.reshape(1, dout))
    return (o_seq, z_seq)


# speedup vs baseline: 1.2957x; 1.1812x over previous
import functools

import jax
import jax.numpy as jnp
from jax import lax
from jax.experimental import pallas as pl
from jax.experimental.pallas import tpu as pltpu
from jax.experimental.pallas import tpu_sc as plsc

NC = 2
NS = 16
KD = 80
BN = 1000
V_TH = 1.0
T = 4


def _deg_body(dst_hbm, zeros_hbm, degp_hbm, idx_v, ones_v, deg_sh, sem):
    nchunk = dst_hbm.shape[2]
    c = lax.axis_index("c")
    s = lax.axis_index("s")

    @pl.when(s == 0)
    def _():
        pltpu.sync_copy(zeros_hbm, deg_sh)

    for i in range(KD // 16):
        ones_v[pl.ds(i * 16, 16)] = jnp.ones((16,), jnp.float32)
    pltpu.sync_copy(dst_hbm.at[c, s], idx_v)
    plsc.subcore_barrier()

    def body(j, carry):
        pltpu.sync_copy(ones_v, deg_sh.at[idx_v.at[j]], add=True)
        return carry

    lax.fori_loop(0, nchunk, body, 0)
    plsc.subcore_barrier()

    @pl.when(s == 0)
    def _():
        pltpu.sync_copy(deg_sh, degp_hbm.at[c])


def _deg_partials(dst_r, zeros_n, n):
    nchunk = dst_r.shape[2]
    kern = pl.kernel(
        _deg_body,
        out_type=jax.ShapeDtypeStruct((NC, n), jnp.float32),
        mesh=plsc.VectorSubcoreMesh(core_axis_name="c", subcore_axis_name="s"),
        scratch_types=[
            pltpu.VMEM((nchunk, KD), jnp.int32),
            pltpu.VMEM((KD,), jnp.float32),
            pltpu.MemorySpace.VMEM_SHARED((n,), jnp.float32),
            pltpu.SemaphoreType.DMA,
        ],
    )
    return kern(dst_r, zeros_n)


def _spmem_slices(s, n, src_at, dst_at):
    rpt8 = (n // NS // 8) * 8
    tail = n - NS * rpt8
    pltpu.sync_copy(src_at(s * rpt8, rpt8), dst_at(s * rpt8, rpt8))
    if tail:
        @pl.when(s == 0)
        def _():
            pltpu.sync_copy(src_at(NS * rpt8, tail), dst_at(NS * rpt8, tail))


UNROLL = 10


def _scatter_body(g_hbm, src_hbm, dst_hbm, zeros_hbm, accp_hbm,
                  sidx_v, didx_v, rows_v, acc_sh, sems_d, sems_r):
    nch = dst_hbm.shape[2]
    ks = dst_hbm.shape[3]
    np_ = zeros_hbm.shape[0]
    n = accp_hbm.shape[1]
    c = lax.axis_index("c")
    s = lax.axis_index("s")

    _spmem_slices(s, np_, lambda o, l: zeros_hbm.at[pl.ds(o, l)],
                  lambda o, l: acc_sh.at[pl.ds(o, l)])
    pltpu.sync_copy(src_hbm.at[c, s], sidx_v)
    plsc.subcore_barrier()

    def body(i, carry):
        base = i * UNROLL
        cp_g = [None] * UNROLL
        cp_d = [None] * UNROLL
        cp_g[0] = pltpu.async_copy(g_hbm.at[sidx_v.at[base]],
                                   rows_v[0], sems_r[0])
        cp_d[0] = pltpu.async_copy(dst_hbm.at[c, s, base],
                                   didx_v[0], sems_d[0])
        for j in range(UNROLL):
            if j + 1 < UNROLL:
                cp_g[j + 1] = pltpu.async_copy(
                    g_hbm.at[sidx_v.at[base + j + 1]],
                    rows_v[(j + 1) % 2], sems_r[(j + 1) % 2])
                cp_d[j + 1] = pltpu.async_copy(
                    dst_hbm.at[c, s, base + j + 1],
                    didx_v[(j + 1) % 2], sems_d[(j + 1) % 2])
            cp_g[j].wait()
            cp_d[j].wait()
            pltpu.sync_copy(rows_v[j % 2], acc_sh.at[didx_v[j % 2]],
                            add=True)
        return carry

    lax.fori_loop(0, nch // UNROLL, body, 0)
    plsc.subcore_barrier()
    _spmem_slices(s, n, lambda o, l: acc_sh.at[pl.ds(o, l)],
                  lambda o, l: accp_hbm.at[c, pl.ds(o, l)])


def _scatter_partials(g, src_chunks, dst_chunks, zeros_np, n, d):
    np_ = zeros_np.shape[0]
    nch = dst_chunks.shape[2]
    ks = dst_chunks.shape[3]
    kern = pl.kernel(
        _scatter_body,
        out_type=jax.ShapeDtypeStruct((NC, n, d), jnp.float32),
        mesh=plsc.VectorSubcoreMesh(core_axis_name="c", subcore_axis_name="s"),
        scratch_types=[
            pltpu.VMEM((nch, ks), jnp.int32),
            [pltpu.VMEM((ks,), jnp.int32) for _ in range(2)],
            [pltpu.VMEM((ks, d), jnp.float32) for _ in range(2)],
            pltpu.MemorySpace.VMEM_SHARED((np_, d), jnp.float32),
            [pltpu.SemaphoreType.DMA for _ in range(2)],
            [pltpu.SemaphoreType.DMA for _ in range(2)],
        ],
    )
    return kern(g, src_chunks, dst_chunks, zeros_np)


def _dinv_of(degp_blk):
    deg = degp_blk[:, 0:1] + degp_blk[:, 1:2] + 1.0
    return lax.rsqrt(jnp.maximum(deg, 1e-12))


def _mm_scale_body(x_ref, w_ref, degp_ref, g_ref):
    h = jnp.dot(x_ref[...], w_ref[...], preferred_element_type=jnp.float32)
    g_ref[...] = _dinv_of(degp_ref[...]) * h


def _mm_scale(x, w, degp_t):
    n, din = x.shape
    dout = w.shape[1]
    return pl.pallas_call(
        _mm_scale_body,
        grid=(n // BN,),
        in_specs=[
            pl.BlockSpec((BN, din), lambda i: (i, 0)),
            pl.BlockSpec((din, dout), lambda i: (0, 0)),
            pl.BlockSpec((BN, NC), lambda i: (i, 0)),
        ],
        out_specs=pl.BlockSpec((BN, dout), lambda i: (i, 0)),
        out_shape=jax.ShapeDtypeStruct((n, dout), jnp.float32),
    )(x, w, degp_t)


def _if_body(accp_ref, g_ref, degp_ref, b_ref, o_ref, z_ref):
    dinv = _dinv_of(degp_ref[...])
    g = g_ref[...]
    y = dinv * (accp_ref[0] + accp_ref[1] + g) + b_ref[...]
    z = jnp.zeros_like(y)
    for t in range(T):
        z = z + y
        o = (z >= V_TH).astype(jnp.float32)
        z = z * (1.0 - o)
        o_ref[t] = o
        z_ref[t] = z


def _if_dynamics(accp, g, degp_t, b2d):
    n, d = g.shape
    out_sds = jax.ShapeDtypeStruct((T, n, d), jnp.float32)
    return pl.pallas_call(
        _if_body,
        grid=(n // BN,),
        in_specs=[
            pl.BlockSpec((NC, BN, d), lambda i: (0, i, 0)),
            pl.BlockSpec((BN, d), lambda i: (i, 0)),
            pl.BlockSpec((BN, NC), lambda i: (i, 0)),
            pl.BlockSpec((1, d), lambda i: (0, 0)),
        ],
        out_specs=[
            pl.BlockSpec((T, BN, d), lambda i: (0, i, 0)),
            pl.BlockSpec((T, BN, d), lambda i: (0, i, 0)),
        ],
        out_shape=[out_sds, out_sds],
    )(accp, g, degp_t, b2d)


def kernel(x, edge_index, W, b):
    n, din = x.shape
    dout = W.shape[1]
    e = edge_index.shape[1]
    ept = e // (NC * NS)
    nchunk_d = ept // KD
    ks = 128
    nch = ((-(-ept // ks) + UNROLL - 1) // UNROLL) * UNROLL
    np_ = n + 8

    src_t = edge_index[0].reshape(NC, NS, ept)
    dst_t = edge_index[1].reshape(NC, NS, ept)
    src_chunks = jnp.concatenate(
        [src_t, jnp.zeros((NC, NS, nch * ks - ept), jnp.int32)],
        axis=2).reshape(NC, NS, nch, ks)
    dst_chunks = jnp.concatenate(
        [dst_t, jnp.full((NC, NS, nch * ks - ept), n, jnp.int32)],
        axis=2).reshape(NC, NS, nch, ks)

    dst_rd = edge_index[1].reshape(NC, NS, nchunk_d, KD)
    zeros_n = jnp.zeros((n,), jnp.float32)
    zeros_np = jnp.zeros((np_, dout), jnp.float32)

    degp = _deg_partials(dst_rd, zeros_n, n)
    degp_t = degp.T
    g = _mm_scale(x, W, degp_t)
    accp = _scatter_partials(g, src_chunks, dst_chunks, zeros_np, n, dout)
    o_seq, z_seq = _if_dynamics(accp, g, degp_t, b.reshape(1, dout))
    return (o_seq, z_seq)


# speedup vs baseline: 1.6247x; 1.2539x over previous
import functools

import jax
import jax.numpy as jnp
from jax import lax
from jax.experimental import pallas as pl
from jax.experimental.pallas import tpu as pltpu
from jax.experimental.pallas import tpu_sc as plsc

NC = 2
NS = 16
KD = 80
BN = 1000
V_TH = 1.0
T = 4


def _deg_body(dst_hbm, zeros_hbm, degp_hbm, idx_v, ones_v, deg_sh, sem):
    nchunk = dst_hbm.shape[2]
    c = lax.axis_index("c")
    s = lax.axis_index("s")

    @pl.when(s == 0)
    def _():
        pltpu.sync_copy(zeros_hbm, deg_sh)

    for i in range(KD // 16):
        ones_v[pl.ds(i * 16, 16)] = jnp.ones((16,), jnp.float32)
    pltpu.sync_copy(dst_hbm.at[c, s], idx_v)
    plsc.subcore_barrier()

    def body(j, carry):
        pltpu.sync_copy(ones_v, deg_sh.at[idx_v.at[j]], add=True)
        return carry

    lax.fori_loop(0, nchunk, body, 0)
    plsc.subcore_barrier()

    @pl.when(s == 0)
    def _():
        pltpu.sync_copy(deg_sh, degp_hbm.at[c])


def _deg_partials(dst_r, zeros_n, n):
    nchunk = dst_r.shape[2]
    kern = pl.kernel(
        _deg_body,
        out_type=jax.ShapeDtypeStruct((NC, n), jnp.float32),
        mesh=plsc.VectorSubcoreMesh(core_axis_name="c", subcore_axis_name="s"),
        scratch_types=[
            pltpu.VMEM((nchunk, KD), jnp.int32),
            pltpu.VMEM((KD,), jnp.float32),
            pltpu.MemorySpace.VMEM_SHARED((n,), jnp.float32),
            pltpu.SemaphoreType.DMA,
        ],
    )
    return kern(dst_r, zeros_n)


def _spmem_slices(s, n, src_at, dst_at):
    rpt8 = (n // NS // 8) * 8
    tail = n - NS * rpt8
    pltpu.sync_copy(src_at(s * rpt8, rpt8), dst_at(s * rpt8, rpt8))
    if tail:
        @pl.when(s == 0)
        def _():
            pltpu.sync_copy(src_at(NS * rpt8, tail), dst_at(NS * rpt8, tail))


UNROLL = 10


def _scatter_body(g_hbm, src_hbm, dst_hbm, zeros_hbm, accp_hbm,
                  sidx_v, didx_v, rows_v, acc_sh, sems_r):
    nch = dst_hbm.shape[2]
    ks = dst_hbm.shape[3]
    np_ = zeros_hbm.shape[0]
    n = accp_hbm.shape[1]
    c = lax.axis_index("c")
    s = lax.axis_index("s")

    _spmem_slices(s, np_, lambda o, l: zeros_hbm.at[pl.ds(o, l)],
                  lambda o, l: acc_sh.at[pl.ds(o, l)])
    pltpu.sync_copy(src_hbm.at[c, s], sidx_v)
    plsc.subcore_barrier()

    pltpu.sync_copy(dst_hbm.at[c, s], didx_v)

    def body(j, carry):
        pltpu.async_copy(g_hbm.at[sidx_v.at[j]], rows_v, sems_r).wait()
        pltpu.sync_copy(rows_v, acc_sh.at[didx_v.at[j]], add=True)
        return carry

    lax.fori_loop(0, nch, body, 0)
    plsc.subcore_barrier()
    _spmem_slices(s, n, lambda o, l: acc_sh.at[pl.ds(o, l)],
                  lambda o, l: accp_hbm.at[c, pl.ds(o, l)])


def _scatter_partials(g, src_chunks, dst_chunks, zeros_np, n, d):
    np_ = zeros_np.shape[0]
    nch = dst_chunks.shape[2]
    ks = dst_chunks.shape[3]
    kern = pl.kernel(
        _scatter_body,
        out_type=jax.ShapeDtypeStruct((NC, n, d), jnp.float32),
        mesh=plsc.VectorSubcoreMesh(core_axis_name="c", subcore_axis_name="s"),
        scratch_types=[
            pltpu.VMEM((nch, ks), jnp.int32),
            pltpu.VMEM((nch, ks), jnp.int32),
            pltpu.VMEM((ks, d), jnp.float32),
            pltpu.MemorySpace.VMEM_SHARED((np_, d), jnp.float32),
            pltpu.SemaphoreType.DMA,
        ],
    )
    return kern(g, src_chunks, dst_chunks, zeros_np)


def _dinv_of(degp_blk):
    deg = degp_blk[:, 0:1] + degp_blk[:, 1:2] + 1.0
    return lax.rsqrt(jnp.maximum(deg, 1e-12))


def _mm_scale_body(x_ref, w_ref, degp_ref, g_ref):
    h = jnp.dot(x_ref[...], w_ref[...], preferred_element_type=jnp.float32)
    g_ref[...] = _dinv_of(degp_ref[...]) * h


def _mm_scale(x, w, degp_t):
    n, din = x.shape
    dout = w.shape[1]
    return pl.pallas_call(
        _mm_scale_body,
        grid=(n // BN,),
        in_specs=[
            pl.BlockSpec((BN, din), lambda i: (i, 0)),
            pl.BlockSpec((din, dout), lambda i: (0, 0)),
            pl.BlockSpec((BN, NC), lambda i: (i, 0)),
        ],
        out_specs=pl.BlockSpec((BN, dout), lambda i: (i, 0)),
        out_shape=jax.ShapeDtypeStruct((n, dout), jnp.float32),
    )(x, w, degp_t)


def _if_body(accp_ref, g_ref, degp_ref, b_ref, o_ref, z_ref):
    dinv = _dinv_of(degp_ref[...])
    g = g_ref[...]
    y = dinv * (accp_ref[0] + accp_ref[1] + g) + b_ref[...]
    z = jnp.zeros_like(y)
    for t in range(T):
        z = z + y
        o = (z >= V_TH).astype(jnp.float32)
        z = z * (1.0 - o)
        o_ref[t] = o
        z_ref[t] = z


def _if_dynamics(accp, g, degp_t, b2d):
    n, d = g.shape
    out_sds = jax.ShapeDtypeStruct((T, n, d), jnp.float32)
    return pl.pallas_call(
        _if_body,
        grid=(n // BN,),
        in_specs=[
            pl.BlockSpec((NC, BN, d), lambda i: (0, i, 0)),
            pl.BlockSpec((BN, d), lambda i: (i, 0)),
            pl.BlockSpec((BN, NC), lambda i: (i, 0)),
            pl.BlockSpec((1, d), lambda i: (0, 0)),
        ],
        out_specs=[
            pl.BlockSpec((T, BN, d), lambda i: (0, i, 0)),
            pl.BlockSpec((T, BN, d), lambda i: (0, i, 0)),
        ],
        out_shape=[out_sds, out_sds],
    )(accp, g, degp_t, b2d)


def kernel(x, edge_index, W, b):
    n, din = x.shape
    dout = W.shape[1]
    e = edge_index.shape[1]
    ept = e // (NC * NS)
    nchunk_d = ept // KD
    ks = 128
    nch = -(-ept // ks)
    np_ = n + 8

    src_t = edge_index[0].reshape(NC, NS, ept)
    dst_t = edge_index[1].reshape(NC, NS, ept)
    src_chunks = jnp.concatenate(
        [src_t, jnp.zeros((NC, NS, nch * ks - ept), jnp.int32)],
        axis=2).reshape(NC, NS, nch, ks)
    dst_chunks = jnp.concatenate(
        [dst_t, jnp.full((NC, NS, nch * ks - ept), n, jnp.int32)],
        axis=2).reshape(NC, NS, nch, ks)

    dst_rd = edge_index[1].reshape(NC, NS, nchunk_d, KD)
    zeros_n = jnp.zeros((n,), jnp.float32)
    zeros_np = jnp.zeros((np_, dout), jnp.float32)

    degp = _deg_partials(dst_rd, zeros_n, n)
    degp_t = degp.T
    g = _mm_scale(x, W, degp_t)
    accp = _scatter_partials(g, src_chunks, dst_chunks, zeros_np, n, dout)
    o_seq, z_seq = _if_dynamics(accp, g, degp_t, b.reshape(1, dout))
    return (o_seq, z_seq)
